# baseline (speedup 1.0000x reference)
import functools

import jax
import jax.numpy as jnp
from jax import lax
from jax.experimental import pallas as pl
from jax.experimental.pallas import tpu as pltpu
from jax.experimental.pallas import tpu_sc as plsc

_NC = 2
_NS = 16
_NW = _NC * _NS
_L = 16


def _make_sc_kernel(S, B, D, C):
    CB = C * B
    rows_per_w = S * B // _NW
    n_chunks = rows_per_w // CB
    mesh = plsc.VectorSubcoreMesh(core_axis_name="c", subcore_axis_name="s")

    @functools.partial(
        pl.kernel,
        mesh=mesh,
        out_type=jax.ShapeDtypeStruct((S * B, D), jnp.float32),
        scratch_types=[
            pltpu.VMEM((CB, D), jnp.float32),
            pltpu.VMEM((C, D), jnp.float32),
            pltpu.SemaphoreType.DMA,
        ],
    )
    def k(x_hbm, t_hbm, out_hbm, buf, tbuf, sem):
        wid = lax.axis_index("s") * _NC + lax.axis_index("c")
        base = wid * rows_per_w

        def body(i):
            r0 = pl.multiple_of(base + i * CB, CB)
            s0 = pl.multiple_of(wid * (S // _NW) + i * C, C)
            pltpu.sync_copy(t_hbm.at[pl.ds(s0, C)], tbuf)
            pltpu.sync_copy(x_hbm.at[pl.ds(r0, CB)], buf)

            def srow(s):
                def dvec(dv):
                    d0 = dv * _L
                    tv = tbuf[s, pl.ds(d0, _L)]
                    for b in range(B):
                        r = s * B + b
                        buf[r, pl.ds(d0, _L)] = buf[r, pl.ds(d0, _L)] + tv
                pl.loop(0, D // _L, unroll=4)(dvec)

            pl.loop(0, C)(srow)
            pltpu.sync_copy(buf, out_hbm.at[pl.ds(r0, CB)])

        pl.loop(0, n_chunks)(body)

    return k


def kernel(x, table):
    S, B, D = x.shape
    out = _make_sc_kernel(S, B, D, 16)(x.reshape(S * B, D), table[:S])
    return out.reshape(S, B, D)


# speedup vs baseline: 1.5367x; 1.5367x over previous
import functools

import jax
import jax.numpy as jnp
from jax import lax
from jax.experimental import pallas as pl
from jax.experimental.pallas import tpu as pltpu
from jax.experimental.pallas import tpu_sc as plsc

_NC = 2
_NS = 16
_NW = _NC * _NS
_L = 16
_NBUF = 4


def _make_sc_kernel(S, B, D, C):
    per_w = S // _NW
    n_chunks = per_w // C
    mesh = plsc.VectorSubcoreMesh(core_axis_name="c", subcore_axis_name="s")

    @functools.partial(
        pl.kernel,
        mesh=mesh,
        out_type=jax.ShapeDtypeStruct((S, B, D), jnp.float32),
        scratch_types=(
            [pltpu.VMEM((C, B, D), jnp.float32) for _ in range(_NBUF)]
            + [pltpu.VMEM((C, D), jnp.float32) for _ in range(_NBUF)]
            + [pltpu.SemaphoreType.DMA for _ in range(2 * _NBUF)]
        ),
    )
    def k(x_hbm, t_hbm, out_hbm, *scr):
        bufs = scr[:_NBUF]
        tbufs = scr[_NBUF:2 * _NBUF]
        sins = scr[2 * _NBUF:3 * _NBUF]
        souts = scr[3 * _NBUF:]
        wid = lax.axis_index("s") * _NC + lax.axis_index("c")
        base = wid * per_w

        def start_in(i, slot):
            s0 = pl.multiple_of(base + i * C, C)
            pltpu.async_copy(x_hbm.at[pl.ds(s0, C)], bufs[slot], sins[slot])
            pltpu.async_copy(t_hbm.at[pl.ds(s0, C)], tbufs[slot], sins[slot])

        def wait_in(slot):
            pltpu.make_async_copy(x_hbm.at[pl.ds(base, C)], bufs[slot],
                                  sins[slot]).wait()
            pltpu.make_async_copy(t_hbm.at[pl.ds(base, C)], tbufs[slot],
                                  sins[slot]).wait()

        def wait_out(slot):
            pltpu.make_async_copy(bufs[slot], out_hbm.at[pl.ds(base, C)],
                                  souts[slot]).wait()

        start_in(0, 0)
        start_in(1, 1)

        def ring(g):
            for b in range(_NBUF):
                i = g + b
                slot = b
                pre = (b + 2) % _NBUF

                @pl.when(i >= 2)
                def _():
                    wait_out(pre)

                @pl.when(i + 2 < n_chunks)
                def _():
                    start_in(i + 2, pre)

                wait_in(slot)

                buf, tbuf = bufs[slot], tbufs[slot]
                for s in range(C):
                    def dvec(dv, s=s, buf=buf, tbuf=tbuf):
                        d0 = dv * _L
                        tv = tbuf[s, pl.ds(d0, _L)]
                        for bb in range(B):
                            buf[s, bb, pl.ds(d0, _L)] = (
                                buf[s, bb, pl.ds(d0, _L)] + tv)
                    pl.loop(0, D // _L, unroll=4)(dvec)

                s0 = pl.multiple_of(base + i * C, C)
                pltpu.async_copy(buf, out_hbm.at[pl.ds(s0, C)], souts[slot])

        pl.loop(0, n_chunks, step=_NBUF)(ring)

        wait_out((n_chunks - 2) % _NBUF)
        wait_out((n_chunks - 1) % _NBUF)

    return k


def kernel(x, table):
    S, B, D = x.shape
    return _make_sc_kernel(S, B, D, 4)(x, table[:S])


# speedup vs baseline: 4.2033x; 2.7352x over previous
import functools

import jax
import jax.numpy as jnp
from jax import lax
from jax.experimental import pallas as pl
from jax.experimental.pallas import tpu as pltpu
from jax.experimental.pallas import tpu_sc as plsc

_NC = 2
_NS = 16
_NW = _NC * _NS
_L = 16
_NBUF = 4


def _make_sc_kernel(S, B, D, C):
    per_w = S // _NW
    n_chunks = per_w // C
    mesh = plsc.VectorSubcoreMesh(core_axis_name="c", subcore_axis_name="s")

    @functools.partial(
        pl.kernel,
        mesh=mesh,
        out_type=jax.ShapeDtypeStruct((S, B, D), jnp.float32),
        scratch_types=(
            [pltpu.VMEM((C, B, D), jnp.float32) for _ in range(_NBUF)]
            + [pltpu.VMEM((C, D), jnp.float32) for _ in range(_NBUF)]
            + [pltpu.SemaphoreType.DMA for _ in range(2 * _NBUF)]
        ),
    )
    def k(x_hbm, t_hbm, out_hbm, *scr):
        bufs = scr[:_NBUF]
        tbufs = scr[_NBUF:2 * _NBUF]
        sins = scr[2 * _NBUF:3 * _NBUF]
        souts = scr[3 * _NBUF:]
        wid = lax.axis_index("s") * _NC + lax.axis_index("c")
        base = wid * per_w

        def start_in(i, slot):
            s0 = pl.multiple_of(base + i * C, C)
            pltpu.async_copy(x_hbm.at[pl.ds(s0, C)], bufs[slot], sins[slot])
            pltpu.async_copy(t_hbm.at[pl.ds(s0, C)], tbufs[slot], sins[slot])

        def wait_in(slot):
            pltpu.make_async_copy(x_hbm.at[pl.ds(base, C)], bufs[slot],
                                  sins[slot]).wait()
            pltpu.make_async_copy(t_hbm.at[pl.ds(base, C)], tbufs[slot],
                                  sins[slot]).wait()

        def wait_out(slot):
            pltpu.make_async_copy(bufs[slot], out_hbm.at[pl.ds(base, C)],
                                  souts[slot]).wait()

        start_in(0, 0)
        start_in(1, 1)

        def ring(g):
            for b in range(_NBUF):
                i = g + b
                slot = b
                pre = (b + 2) % _NBUF

                @pl.when(i >= 2)
                def _():
                    wait_out(pre)

                @pl.when(i + 2 < n_chunks)
                def _():
                    start_in(i + 2, pre)

                wait_in(slot)

                buf, tbuf = bufs[slot], tbufs[slot]
                for s in range(C):
                    def dvec(dv, s=s, buf=buf, tbuf=tbuf):
                        d0 = dv * _L
                        tv = tbuf[s, pl.ds(d0, _L)]
                        for bb in range(B):
                            plsc.addupdate(buf.at[s, bb, pl.ds(d0, _L)], tv)
                    pl.loop(0, D // _L, unroll=8)(dvec)

                s0 = pl.multiple_of(base + i * C, C)
                pltpu.async_copy(buf, out_hbm.at[pl.ds(s0, C)], souts[slot])

        pl.loop(0, n_chunks, step=_NBUF)(ring)

        wait_out((n_chunks - 2) % _NBUF)
        wait_out((n_chunks - 1) % _NBUF)

    return k


def kernel(x, table):
    S, B, D = x.shape
    return _make_sc_kernel(S, B, D, 4)(x, table[:S])
